# Initial kernel scaffold; baseline (speedup 1.0000x reference)
import jax
import jax.numpy as jnp
from jax.experimental import pallas as pl


def kernel(subword_embeddings, word_to_subword_mapping, W_proj, b_proj, W_score, b_score):
    raise NotImplementedError("write your pallas kernel here")


# speedup vs baseline: 6.4257x; 6.4257x over previous
import functools

import jax
import jax.numpy as jnp
from jax import lax
from jax.experimental import pallas as pl
from jax.experimental.pallas import tpu as pltpu
from jax.experimental.pallas import tpu_sc as plsc

_T = 16384
_W = 8192
_H = 768

_NT = 32
_WPT = _W // _NT
_WG = 64
_NG = _WPT // _WG
_RC = 32
_NC = _H // 16
_BSL = _WPT + 16


def _scores_body(e_ref, wp_ref, bp_ref, ws_ref, w_ref):
    p = jnp.tanh(
        jnp.dot(e_ref[...], wp_ref[...], preferred_element_type=jnp.float32)
        + bp_ref[...]
    )
    s = jnp.sum(p * ws_ref[...], axis=1)
    w_ref[...] = jnp.exp(s)


def _weights(E, W_proj, b_proj, W_score):
    blk = 1024
    grid = (_T // blk,)
    return pl.pallas_call(
        _scores_body,
        grid=grid,
        in_specs=[
            pl.BlockSpec((blk, _H), lambda i: (i, 0)),
            pl.BlockSpec((_H, _H), lambda i: (0, 0)),
            pl.BlockSpec((1, _H), lambda i: (0, 0)),
            pl.BlockSpec((1, _H), lambda i: (0, 0)),
        ],
        out_specs=pl.BlockSpec((blk,), lambda i: (i,)),
        out_shape=jax.ShapeDtypeStruct((_T,), jnp.float32),
    )(E, W_proj, b_proj, W_score)


def _sc_pool_body(e_hbm, w_hbm, bnd_hbm, out_hbm, bnd_v, rows_v, wv, out_v, den_v):
    wid = lax.axis_index("s") * 2 + lax.axis_index("c")
    wlo = wid * _WPT
    pltpu.sync_copy(bnd_hbm.at[pl.ds(wlo, _BSL)], bnd_v)

    def _bnd_at(q):
        return bnd_v[pl.ds(q, 16)][0]

    def _count_le(x):
        def sbody(_, carry):
            lo, hi = carry
            mid = (lo + hi) // 2
            le = _bnd_at(mid) <= x
            return (jnp.where(le, mid, lo), jnp.where(le, hi, mid))
        lo, _hi = lax.fori_loop(
            0, 9, sbody, (jnp.int32(-1), jnp.int32(_BSL - 16 + 1))
        )
        return lo + 1

    for g in range(_NG):
        def _zero(j, _):
            out_v[j // _NC, pl.ds((j % _NC) * 16, 16)] = jnp.zeros((16,), jnp.float32)
            return _
        lax.fori_loop(0, _WG * _NC, _zero, 0)

        def _zero_d(j, _):
            den_v[pl.ds(j * 16, 16)] = jnp.zeros((16,), jnp.float32)
            return _
        lax.fori_loop(0, _WG, _zero_d, 0)

        s_grp = _bnd_at(g * _WG)
        e_grp = _bnd_at((g + 1) * _WG)
        s8 = (s_grp // 8) * 8
        nch = (e_grp - s8 + _RC - 1) // _RC

        def _chunk(k, p):
            a = s8 + k * _RC
            r_load = jnp.minimum(a, _T - _RC)
            pltpu.sync_copy(e_hbm.at[pl.ds(r_load, _RC)], rows_v)
            pltpu.sync_copy(w_hbm.at[pl.ds(r_load, _RC)], wv.at[pl.ds(0, _RC)])
            bc = jnp.minimum(a + _RC, e_grp)
            q = _count_le(bc - 1) - 1

            def _word(j, _):
                b0 = _bnd_at(j)
                b1 = _bnd_at(j + 1)
                lo = jnp.maximum(b0, a)
                hi = jnp.minimum(b1, bc)
                orow = j - g * _WG

                def _row(t, _):
                    i = t - r_load
                    wt = wv[pl.ds(i, 16)][0]
                    wtv = jnp.full((16,), wt, jnp.float32)
                    for c in range(_NC):
                        v = rows_v[i, pl.ds(c * 16, 16)]
                        out_v[orow, pl.ds(c * 16, 16)] += v * wtv
                    den_v[pl.ds(orow * 16, 16)] += wtv
                    return _

                lax.fori_loop(lo, hi, _row, 0)
                return _

            lax.fori_loop(p, q + 1, _word, 0)
            return jnp.maximum(q, p)

        lax.fori_loop(0, nch, _chunk, jnp.int32(g * _WG))

        def _div(j, _):
            d = den_v[pl.ds(j * 16, 16)]
            inv = jnp.where(d > 0.0, 1.0 / d, 0.0)
            for c in range(_NC):
                out_v[j, pl.ds(c * 16, 16)] = out_v[j, pl.ds(c * 16, 16)] * inv
            return _
        lax.fori_loop(0, _WG, _div, 0)

        pltpu.sync_copy(out_v, out_hbm.at[pl.ds(wlo + g * _WG, _WG)])


_sc_pool = functools.partial(
    pl.kernel,
    out_type=jax.ShapeDtypeStruct((_W, _H), jnp.float32),
    mesh=plsc.VectorSubcoreMesh(
        core_axis_name="c", subcore_axis_name="s", num_cores=2, num_subcores=16
    ),
    scratch_types=[
        pltpu.VMEM((_BSL,), jnp.int32),
        pltpu.VMEM((_RC, _H), jnp.float32),
        pltpu.VMEM((_RC + 16,), jnp.float32),
        pltpu.VMEM((_WG, _H), jnp.float32),
        pltpu.VMEM((_WG * 16,), jnp.float32),
    ],
)(_sc_pool_body)


def kernel(subword_embeddings, word_to_subword_mapping, W_proj, b_proj, W_score, b_score):
    E = subword_embeddings
    w = _weights(E, W_proj, b_proj.reshape(1, _H), W_score.reshape(1, _H))
    starts = word_to_subword_mapping[:, 0].astype(jnp.int32)
    bnd = jnp.concatenate([starts, jnp.full((16,), _T, jnp.int32)])
    return _sc_pool(E, w, bnd)


# speedup vs baseline: 16.0078x; 2.4912x over previous
import functools

import jax
import jax.numpy as jnp
from jax import lax
from jax.experimental import pallas as pl
from jax.experimental.pallas import tpu as pltpu
from jax.experimental.pallas import tpu_sc as plsc

_T = 16384
_W = 8192
_H = 768

_NT = 32
_WPT = _W // _NT
_WG = 64
_NG = _WPT // _WG
_RC = 32
_NC = _H // 16
_BSL = _WPT + 16


def _scores_body(e_ref, wp_ref, bp_ref, ws_ref, w_ref):
    p = jnp.tanh(
        jnp.dot(
            e_ref[...].astype(jnp.bfloat16),
            wp_ref[...].astype(jnp.bfloat16),
            preferred_element_type=jnp.float32,
        )
        + bp_ref[...]
    )
    s = jnp.sum(p * ws_ref[...], axis=1)
    w_ref[...] = jnp.exp(s)


def _weights(E, W_proj, b_proj, W_score):
    blk = 1024
    grid = (_T // blk,)
    return pl.pallas_call(
        _scores_body,
        grid=grid,
        in_specs=[
            pl.BlockSpec((blk, _H), lambda i: (i, 0)),
            pl.BlockSpec((_H, _H), lambda i: (0, 0)),
            pl.BlockSpec((1, _H), lambda i: (0, 0)),
            pl.BlockSpec((1, _H), lambda i: (0, 0)),
        ],
        out_specs=pl.BlockSpec((blk,), lambda i: (i,)),
        out_shape=jax.ShapeDtypeStruct((_T,), jnp.float32),
    )(E, W_proj, b_proj, W_score)


def _sc_pool_body(
    e_hbm, w_hbm, bnd_hbm, out_hbm,
    bnd_v, rows_a, rows_b, wv_a, wv_b, out_v, den_v,
    sem_ra, sem_wa, sem_rb, sem_wb,
):
    wid = lax.axis_index("s") * 2 + lax.axis_index("c")
    wlo = wid * _WPT
    pltpu.sync_copy(bnd_hbm.at[pl.ds(wlo, _BSL)], bnd_v)

    def _bnd_at(q):
        return bnd_v[pl.ds(q, 16)][0]

    def _count_le(x):
        def sbody(_, carry):
            lo, hi = carry
            mid = (lo + hi) // 2
            le = _bnd_at(mid) <= x
            return (jnp.where(le, mid, lo), jnp.where(le, hi, mid))
        lo, _hi = lax.fori_loop(
            0, 9, sbody, (jnp.int32(-1), jnp.int32(_BSL - 16 + 1))
        )
        return lo + 1

    def _start(kc, s8, nch, rows_buf, wv_buf, sem_r, sem_w):
        kcc = jnp.minimum(kc, jnp.maximum(nch - 1, 0))
        r_load = jnp.minimum(s8 + kcc * _RC, _T - _RC)
        pltpu.async_copy(e_hbm.at[pl.ds(r_load, _RC)], rows_buf, sem_r)
        pltpu.async_copy(w_hbm.at[pl.ds(r_load, _RC)], wv_buf.at[pl.ds(0, _RC)], sem_w)

    def _wait(rows_buf, wv_buf, sem_r, sem_w):
        pltpu.make_async_copy(e_hbm.at[pl.ds(0, _RC)], rows_buf, sem_r).wait()
        pltpu.make_async_copy(
            w_hbm.at[pl.ds(0, _RC)], wv_buf.at[pl.ds(0, _RC)], sem_w
        ).wait()

    for g in range(_NG):
        s_grp = _bnd_at(g * _WG)
        e_grp = _bnd_at((g + 1) * _WG)
        s8 = (s_grp // 8) * 8
        nch = (e_grp - s8 + _RC - 1) // _RC

        _start(jnp.int32(0), s8, nch, rows_a, wv_a, sem_ra, sem_wa)

        def _zero(j, _):
            for c in range(_NC):
                out_v[j, pl.ds(c * 16, 16)] = jnp.zeros((16,), jnp.float32)
            den_v[pl.ds(j * 16, 16)] = jnp.zeros((16,), jnp.float32)
            return _
        lax.fori_loop(0, _WG, _zero, 0)

        def _process(kc, rows_buf, wv_buf, p):
            a = s8 + jnp.minimum(kc, jnp.maximum(nch - 1, 0)) * _RC
            r_load = jnp.minimum(a, _T - _RC)
            bc = jnp.where(kc < nch, jnp.minimum(a + _RC, e_grp), a)
            q = _count_le(bc - 1) - 1

            def _word(j, b0):
                b1 = _bnd_at(j + 1)
                lo = jnp.maximum(b0, a)
                hi = jnp.minimum(b1, bc)
                orow = j - g * _WG

                @plsc.parallel_loop(lo, hi, step=2)
                def _row(t):
                    i = t - r_load
                    wt = wv_buf[pl.ds(i, 16)][0]
                    wtv = jnp.full((16,), wt, jnp.float32)
                    has2 = (t + 1) < hi
                    i2 = jnp.where(has2, i + 1, i)
                    wt2 = wv_buf[pl.ds(i2, 16)][0]
                    wtv2 = jnp.full(
                        (16,), jnp.where(has2, wt2, jnp.float32(0)), jnp.float32
                    )
                    nblk = _NC // 8
                    prev = None
                    for blk in range(nblk):
                        cur = (blk, [
                            rows_buf[i, pl.ds((blk * 8 + c) * 16, 16)] * wtv
                            + rows_buf[i2, pl.ds((blk * 8 + c) * 16, 16)] * wtv2
                            for c in range(8)
                        ])
                        if prev is not None:
                            pb, pv = prev
                            for c in range(8):
                                plsc.addupdate(
                                    out_v.at[orow, pl.ds((pb * 8 + c) * 16, 16)],
                                    pv[c],
                                )
                        prev = cur
                    pb, pv = prev
                    for c in range(8):
                        plsc.addupdate(
                            out_v.at[orow, pl.ds((pb * 8 + c) * 16, 16)], pv[c]
                        )
                    plsc.addupdate(den_v.at[pl.ds(orow * 16, 16)], wtv + wtv2)
                return b1

            lax.fori_loop(p, q + 1, _word, _bnd_at(p))
            return jnp.maximum(q, p)

        nch2 = (nch + 1) // 2

        def _pair(kk, p):
            _start(2 * kk + 1, s8, nch, rows_b, wv_b, sem_rb, sem_wb)
            _wait(rows_a, wv_a, sem_ra, sem_wa)
            p = _process(2 * kk, rows_a, wv_a, p)
            _start(2 * kk + 2, s8, nch, rows_a, wv_a, sem_ra, sem_wa)
            _wait(rows_b, wv_b, sem_rb, sem_wb)
            p = _process(2 * kk + 1, rows_b, wv_b, p)
            return p

        lax.fori_loop(0, nch2, _pair, jnp.int32(g * _WG))
        _wait(rows_a, wv_a, sem_ra, sem_wa)

        def _div(j, _):
            d = den_v[pl.ds(j * 16, 16)]
            inv = jnp.where(d > 0.0, 1.0 / d, 0.0)
            for c in range(_NC):
                out_v[j, pl.ds(c * 16, 16)] = out_v[j, pl.ds(c * 16, 16)] * inv
            return _
        lax.fori_loop(0, _WG, _div, 0)

        pltpu.sync_copy(out_v, out_hbm.at[pl.ds(wlo + g * _WG, _WG)])


_sc_pool = functools.partial(
    pl.kernel,
    out_type=jax.ShapeDtypeStruct((_W, _H), jnp.float32),
    mesh=plsc.VectorSubcoreMesh(
        core_axis_name="c", subcore_axis_name="s", num_cores=2, num_subcores=16
    ),
    scratch_types=[
        pltpu.VMEM((_BSL,), jnp.int32),
        pltpu.VMEM((_RC, _H), jnp.float32),
        pltpu.VMEM((_RC, _H), jnp.float32),
        pltpu.VMEM((_RC + 16,), jnp.float32),
        pltpu.VMEM((_RC + 16,), jnp.float32),
        pltpu.VMEM((_WG, _H), jnp.float32),
        pltpu.VMEM((_WG * 16,), jnp.float32),
        pltpu.SemaphoreType.DMA,
        pltpu.SemaphoreType.DMA,
        pltpu.SemaphoreType.DMA,
        pltpu.SemaphoreType.DMA,
    ],
)(_sc_pool_body)


def kernel(subword_embeddings, word_to_subword_mapping, W_proj, b_proj, W_score, b_score):
    E = subword_embeddings
    w = _weights(E, W_proj, b_proj.reshape(1, _H), W_score.reshape(1, _H))
    starts = word_to_subword_mapping[:, 0].astype(jnp.int32)
    bnd = jnp.concatenate([starts, jnp.full((16,), _T, jnp.int32)])
    return _sc_pool(E, w, bnd)


# speedup vs baseline: 16.6299x; 1.0389x over previous
import functools

import jax
import jax.numpy as jnp
from jax import lax
from jax.experimental import pallas as pl
from jax.experimental.pallas import tpu as pltpu
from jax.experimental.pallas import tpu_sc as plsc

_T = 16384
_W = 8192
_H = 768

_NT = 32
_WPT = _W // _NT
_WG = 64
_NG = _WPT // _WG
_RC = 32
_NC = _H // 16
_BSL = _WPT + 16


def _scores_body(e_ref, wp_ref, bp_ref, ws_ref, s_ref):
    p = jnp.tanh(
        jnp.dot(
            e_ref[...].astype(jnp.bfloat16),
            wp_ref[...].astype(jnp.bfloat16),
            preferred_element_type=jnp.float32,
        )
        + bp_ref[...]
    )
    s_ref[...] = jnp.dot(
        p.astype(jnp.bfloat16),
        ws_ref[...].astype(jnp.bfloat16),
        preferred_element_type=jnp.float32,
    )


def _weights(E, W_proj, b_proj, W_score_rep):
    blk = 1024
    grid = (_T // blk,)
    return pl.pallas_call(
        _scores_body,
        grid=grid,
        in_specs=[
            pl.BlockSpec((blk, _H), lambda i: (i, 0)),
            pl.BlockSpec((_H, _H), lambda i: (0, 0)),
            pl.BlockSpec((1, _H), lambda i: (0, 0)),
            pl.BlockSpec((_H, 128), lambda i: (0, 0)),
        ],
        out_specs=pl.BlockSpec((blk, 128), lambda i: (i, 0)),
        out_shape=jax.ShapeDtypeStruct((_T, 128), jnp.float32),
    )(E, W_proj, b_proj, W_score_rep)


def _sc_pool_body(
    e_hbm, w_hbm, bnd_hbm, out_hbm,
    bnd_v, rows_a, rows_b, wv_a, wv_b, out_v, den_v,
    sem_ra, sem_wa, sem_rb, sem_wb,
):
    wid = lax.axis_index("s") * 2 + lax.axis_index("c")
    wlo = wid * _WPT
    pltpu.sync_copy(bnd_hbm.at[pl.ds(wlo, _BSL)], bnd_v)

    def _bnd_at(q):
        return bnd_v[pl.ds(q, 16)][0]

    def _count_le(x):
        def sbody(_, carry):
            lo, hi = carry
            mid = (lo + hi) // 2
            le = _bnd_at(mid) <= x
            return (jnp.where(le, mid, lo), jnp.where(le, hi, mid))
        lo, _hi = lax.fori_loop(
            0, 9, sbody, (jnp.int32(-1), jnp.int32(_BSL - 16 + 1))
        )
        return lo + 1

    def _start(kc, s8, nch, rows_buf, wv_buf, sem_r, sem_w):
        kcc = jnp.minimum(kc, jnp.maximum(nch - 1, 0))
        r_load = jnp.minimum(s8 + kcc * _RC, _T - _RC)
        pltpu.async_copy(e_hbm.at[pl.ds(r_load, _RC)], rows_buf, sem_r)
        pltpu.async_copy(w_hbm.at[pl.ds(r_load, _RC)], wv_buf, sem_w)

    def _wait(rows_buf, wv_buf, sem_r, sem_w):
        pltpu.make_async_copy(e_hbm.at[pl.ds(0, _RC)], rows_buf, sem_r).wait()
        pltpu.make_async_copy(w_hbm.at[pl.ds(0, _RC)], wv_buf, sem_w).wait()

    for g in range(_NG):
        s_grp = _bnd_at(g * _WG)
        e_grp = _bnd_at((g + 1) * _WG)
        s8 = (s_grp // 8) * 8
        nch = (e_grp - s8 + _RC - 1) // _RC

        _start(jnp.int32(0), s8, nch, rows_a, wv_a, sem_ra, sem_wa)

        def _zero(j, _):
            for c in range(_NC):
                out_v[j, pl.ds(c * 16, 16)] = jnp.zeros((16,), jnp.float32)
            den_v[pl.ds(j * 16, 16)] = jnp.zeros((16,), jnp.float32)
            return _
        lax.fori_loop(0, _WG, _zero, 0)

        def _process(kc, rows_buf, wv_buf, p):
            a = s8 + jnp.minimum(kc, jnp.maximum(nch - 1, 0)) * _RC
            r_load = jnp.minimum(a, _T - _RC)
            bc = jnp.where(kc < nch, jnp.minimum(a + _RC, e_grp), a)
            q = _count_le(bc - 1) - 1

            def _word(j, b0):
                b1 = _bnd_at(j + 1)
                lo = jnp.maximum(b0, a)
                hi = jnp.minimum(b1, bc)
                orow = j - g * _WG

                @plsc.parallel_loop(lo, hi, step=2)
                def _row(t):
                    i = t - r_load
                    wtv = jnp.exp(wv_buf[i, pl.ds(0, 16)])
                    has2 = (t + 1) < hi
                    i2 = jnp.where(has2, i + 1, i)
                    wtv2 = jnp.exp(wv_buf[i2, pl.ds(0, 16)]) * jnp.full(
                        (16,), jnp.where(has2, jnp.float32(1), jnp.float32(0)),
                        jnp.float32,
                    )
                    nblk = _NC // 8
                    prev = None
                    for blk in range(nblk):
                        cur = (blk, [
                            rows_buf[i, pl.ds((blk * 8 + c) * 16, 16)] * wtv
                            + rows_buf[i2, pl.ds((blk * 8 + c) * 16, 16)] * wtv2
                            for c in range(8)
                        ])
                        if prev is not None:
                            pb, pv = prev
                            for c in range(8):
                                plsc.addupdate(
                                    out_v.at[orow, pl.ds((pb * 8 + c) * 16, 16)],
                                    pv[c],
                                )
                        prev = cur
                    pb, pv = prev
                    for c in range(8):
                        plsc.addupdate(
                            out_v.at[orow, pl.ds((pb * 8 + c) * 16, 16)], pv[c]
                        )
                    plsc.addupdate(den_v.at[pl.ds(orow * 16, 16)], wtv + wtv2)
                return b1

            lax.fori_loop(p, q + 1, _word, _bnd_at(p))
            return jnp.maximum(q, p)

        nch2 = (nch + 1) // 2

        def _pair(kk, p):
            _start(2 * kk + 1, s8, nch, rows_b, wv_b, sem_rb, sem_wb)
            _wait(rows_a, wv_a, sem_ra, sem_wa)
            p = _process(2 * kk, rows_a, wv_a, p)
            _start(2 * kk + 2, s8, nch, rows_a, wv_a, sem_ra, sem_wa)
            _wait(rows_b, wv_b, sem_rb, sem_wb)
            p = _process(2 * kk + 1, rows_b, wv_b, p)
            return p

        lax.fori_loop(0, nch2, _pair, jnp.int32(g * _WG))
        _wait(rows_a, wv_a, sem_ra, sem_wa)

        def _div(j, _):
            d = den_v[pl.ds(j * 16, 16)]
            inv = jnp.where(d > 0.0, 1.0 / d, 0.0)
            for c in range(_NC):
                out_v[j, pl.ds(c * 16, 16)] = out_v[j, pl.ds(c * 16, 16)] * inv
            return _
        lax.fori_loop(0, _WG, _div, 0)

        pltpu.sync_copy(out_v, out_hbm.at[pl.ds(wlo + g * _WG, _WG)])


_sc_pool = functools.partial(
    pl.kernel,
    out_type=jax.ShapeDtypeStruct((_W, _H), jnp.float32),
    mesh=plsc.VectorSubcoreMesh(
        core_axis_name="c", subcore_axis_name="s", num_cores=2, num_subcores=16
    ),
    scratch_types=[
        pltpu.VMEM((_BSL,), jnp.int32),
        pltpu.VMEM((_RC, _H), jnp.float32),
        pltpu.VMEM((_RC, _H), jnp.float32),
        pltpu.VMEM((_RC, 128), jnp.float32),
        pltpu.VMEM((_RC, 128), jnp.float32),
        pltpu.VMEM((_WG, _H), jnp.float32),
        pltpu.VMEM((_WG * 16,), jnp.float32),
        pltpu.SemaphoreType.DMA,
        pltpu.SemaphoreType.DMA,
        pltpu.SemaphoreType.DMA,
        pltpu.SemaphoreType.DMA,
    ],
)(_sc_pool_body)


def kernel(subword_embeddings, word_to_subword_mapping, W_proj, b_proj, W_score, b_score):
    E = subword_embeddings
    ws_rep = jnp.broadcast_to(W_score.reshape(_H, 1), (_H, 128))
    s2d = _weights(E, W_proj, b_proj.reshape(1, _H), ws_rep)
    starts = word_to_subword_mapping[:, 0].astype(jnp.int32)
    bnd = jnp.concatenate([starts, jnp.full((16,), _T, jnp.int32)])
    return _sc_pool(E, s2d, bnd)
